# baseline (speedup 1.0000x reference)
import functools

import jax
import jax.numpy as jnp
from jax import lax
from jax.experimental import pallas as pl
from jax.experimental.pallas import tpu as pltpu
from jax.experimental.pallas import tpu_sc as plsc

F32 = jnp.float32
I32 = jnp.int32

HID = 128
TW = 256
SW = 144
NC = 2
NS = 16
NW = NC * NS
L = 16

CH = 80
EB = 1600
NB = 1000
PACK = 32


def _silu(x):
    return x * jax.nn.sigmoid(x)


def _tables_body(h_ref, c_ref, lb_ref, wa_ref, wb_ref, b1_ref,
                 ta_ref, tb_ref):
    h = h_ref[:]
    c = c_ref[:]
    nb = h.shape[0]
    a = jnp.dot(h, wa_ref[:], preferred_element_type=F32) + b1_ref[:]
    b = jnp.dot(h, wb_ref[:], preferred_element_type=F32)
    ta_ref[:] = jnp.concatenate(
        [a, c, lb_ref[:], jnp.zeros((nb, TW - HID - 4), F32)], axis=1)
    tb_ref[:] = jnp.concatenate(
        [b, -c, jnp.zeros((nb, TW - HID - 3), F32)], axis=1)


def _build_tables(h, coords, lbn, wa, wb, b1):
    n = h.shape[0]
    return pl.pallas_call(
        _tables_body,
        grid=(n // NB,),
        in_specs=[
            pl.BlockSpec((NB, HID), lambda i: (i, 0)),
            pl.BlockSpec((NB, 3), lambda i: (i, 0)),
            pl.BlockSpec((NB, 1), lambda i: (i, 0)),
            pl.BlockSpec((HID, HID), lambda i: (0, 0)),
            pl.BlockSpec((HID, HID), lambda i: (0, 0)),
            pl.BlockSpec((1, HID), lambda i: (0, 0)),
        ],
        out_specs=[
            pl.BlockSpec((NB, TW), lambda i: (i, 0)),
            pl.BlockSpec((NB, TW), lambda i: (i, 0)),
        ],
        out_shape=[
            jax.ShapeDtypeStruct((n, TW), F32),
            jax.ShapeDtypeStruct((n, TW), F32),
        ],
    )(h, coords, lbn, wa, wb, b1)


def _gather_sum(ta, tb, row, col, e):
    epw = e // NW
    nchunk = epw // CH
    mesh = plsc.VectorSubcoreMesh(core_axis_name="c", subcore_axis_name="s")

    @functools.partial(
        pl.kernel,
        out_type=jax.ShapeDtypeStruct((e, SW), F32),
        mesh=mesh,
        scratch_types=[
            pltpu.VMEM((2, CH), I32),
            pltpu.VMEM((2, CH), I32),
            pltpu.VMEM((2, CH, TW), F32),
            pltpu.VMEM((2, CH, TW), F32),
            pltpu.VMEM((CH, SW), F32),
            pltpu.SemaphoreType.DMA,
            pltpu.SemaphoreType.DMA,
            pltpu.SemaphoreType.DMA,
            pltpu.SemaphoreType.DMA,
        ],
    )
    def k(ta_hbm, tb_hbm, row_hbm, col_hbm, out_hbm,
          idxa, idxb, bufa, bufb, sbuf, sa0, sb0, sa1, sb1):
        wid = lax.axis_index("s") * NC + lax.axis_index("c")
        base = wid * epw
        sems = ((sa0, sb0), (sa1, sb1))

        def start(i, s):
            off = base + i * CH
            pltpu.sync_copy(row_hbm.at[pl.ds(off, CH)], idxa.at[s])
            pltpu.sync_copy(col_hbm.at[pl.ds(off, CH)], idxb.at[s])
            pltpu.async_copy(ta_hbm.at[idxa.at[s]], bufa.at[s], sems[s][0])
            pltpu.async_copy(tb_hbm.at[idxb.at[s]], bufb.at[s], sems[s][1])

        def finish(i, s):
            pltpu.make_async_copy(
                ta_hbm.at[idxa.at[s]], bufa.at[s], sems[s][0]).wait()
            pltpu.make_async_copy(
                tb_hbm.at[idxb.at[s]], bufb.at[s], sems[s][1]).wait()

            def add_row(j, c2):
                for t in range(SW // L):
                    sl = pl.ds(t * L, L)
                    sbuf[j, sl] = bufa[s, j, sl] + bufb[s, j, sl]
                return c2

            lax.fori_loop(0, CH, add_row, 0)
            pltpu.sync_copy(sbuf, out_hbm.at[pl.ds(base + i * CH, CH)])

        assert nchunk % 2 == 1
        start(0, 0)

        def body(g, carry):
            i0 = 2 * g
            start(i0 + 1, 1)
            finish(i0, 0)
            start(i0 + 2, 0)
            finish(i0 + 1, 1)
            return carry

        lax.fori_loop(0, (nchunk - 1) // 2, body, 0)
        finish(nchunk - 1, 0)

    return k(ta, tb, row, col)


def _edge_body(s_ref, w2_ref, b2_ref, cw1_ref, cb1_ref, cw2_ref, cb2_ref,
               wr_ref, oh_ref, oc_ref):
    s = s_ref[:]
    pre_h = s[:, :HID]
    d3 = s[:, HID:HID + 3]
    lb = s[:, HID + 3:HID + 4].astype(I32)
    r2 = jnp.sum(d3 * d3, axis=1, keepdims=True)
    r = jnp.sqrt(r2)
    pre = pre_h + r * wr_ref[:]
    act = _silu(pre)
    msg = jnp.dot(act, w2_ref[:], preferred_element_type=F32) + b2_ref[:]
    t = _silu(jnp.dot(msg, cw1_ref[:], preferred_element_type=F32) + cb1_ref[:])
    cw = jnp.sum(t * cw2_ref[:], axis=1, keepdims=True) + cb2_ref[:]
    upd = cw * d3 / (r + 1e-8)
    oh_ref[:] = msg
    lane = lax.broadcasted_iota(I32, (s.shape[0], HID), 1)
    oc = jnp.zeros((s.shape[0], HID), F32)
    for c in range(3):
        oc = oc + jnp.where(lane == lb + c, upd[:, c:c + 1], 0.0)
    oc_ref[:] = oc


def _edge_mlp(s, w2, b2, cw1, cb1, cw2r, cb2, wr, e):
    return pl.pallas_call(
        _edge_body,
        grid=(e // EB,),
        in_specs=[
            pl.BlockSpec((EB, SW), lambda i: (i, 0)),
            pl.BlockSpec((HID, HID), lambda i: (0, 0)),
            pl.BlockSpec((1, HID), lambda i: (0, 0)),
            pl.BlockSpec((HID, HID // 2), lambda i: (0, 0)),
            pl.BlockSpec((1, HID // 2), lambda i: (0, 0)),
            pl.BlockSpec((1, HID // 2), lambda i: (0, 0)),
            pl.BlockSpec((1, 1), lambda i: (0, 0)),
            pl.BlockSpec((1, HID), lambda i: (0, 0)),
        ],
        out_specs=[
            pl.BlockSpec((EB, HID), lambda i: (i, 0)),
            pl.BlockSpec((EB, HID), lambda i: (i, 0)),
        ],
        out_shape=[
            jax.ShapeDtypeStruct((e, HID), F32),
            jax.ShapeDtypeStruct((e, HID), F32),
        ],
    )(s, w2, b2, cw1, cb1, cw2r, cb2, wr)


def _scatter_add(oh, oc, row, n, e):
    epw = e // NW
    nchunk = epw // CH
    npad = (n + 8 * NS - 1) // (8 * NS) * (8 * NS)
    rpt = (npad // NS + 7) // 8 * 8
    npad = rpt * NS
    crpt = ((n + PACK - 1) // PACK + NS - 1) // NS
    crpt = (crpt + 7) // 8 * 8
    ncr = crpt * NS
    zr = 128
    mesh = plsc.VectorSubcoreMesh(core_axis_name="c", subcore_axis_name="s")

    @functools.partial(
        pl.kernel,
        out_type=[
            jax.ShapeDtypeStruct((NC, npad, HID), F32),
            jax.ShapeDtypeStruct((NC, ncr, HID), F32),
        ],
        mesh=mesh,
        scratch_types=[
            pltpu.VMEM_SHARED((npad, HID), F32),
            pltpu.VMEM_SHARED((ncr, HID), F32),
            pltpu.VMEM((2, CH), I32),
            pltpu.VMEM((2, CH), I32),
            pltpu.VMEM((2, CH, HID), F32),
            pltpu.VMEM((2, CH, HID), F32),
            pltpu.SemaphoreType.DMA,
            pltpu.SemaphoreType.DMA,
            pltpu.SemaphoreType.DMA,
            pltpu.SemaphoreType.DMA,
            pltpu.SemaphoreType.DMA,
            pltpu.SemaphoreType.DMA,
        ],
    )
    def k(oh_hbm, oc_hbm, row_hbm, outh_hbm, outc_hbm,
          acch, accc, idx, idxc, valh, valc,
          si0, sh0, sc0, si1, sh1, sc1):
        cid = lax.axis_index("c")
        sid = lax.axis_index("s")
        wid = sid * NC + cid
        base = wid * epw
        sems = ((si0, sh0, sc0), (si1, sh1, sc1))

        def zrow(j, c2):
            for t in range(HID // L):
                valh[0, j, pl.ds(t * L, L)] = jnp.zeros((L,), F32)
            return c2

        lax.fori_loop(0, CH, zrow, 0)
        for kk in range((rpt + CH - 1) // CH):
            sz = min(CH, rpt - kk * CH)
            pltpu.sync_copy(valh.at[0].at[pl.ds(0, sz)],
                            acch.at[pl.ds(sid * rpt + kk * CH, sz)])
        pltpu.sync_copy(valh.at[0].at[pl.ds(0, crpt)],
                        accc.at[pl.ds(sid * crpt, crpt)])
        plsc.subcore_barrier()

        def start(i, s):
            off = base + i * CH
            pltpu.async_copy(row_hbm.at[pl.ds(off, CH)], idx.at[s], sems[s][0])
            pltpu.async_copy(oh_hbm.at[pl.ds(off, CH)], valh.at[s], sems[s][1])
            pltpu.async_copy(oc_hbm.at[pl.ds(off, CH)], valc.at[s], sems[s][2])

        def finish(i, s):
            off = base + i * CH
            pltpu.make_async_copy(
                row_hbm.at[pl.ds(off, CH)], idx.at[s], sems[s][0]).wait()
            pltpu.make_async_copy(
                oh_hbm.at[pl.ds(off, CH)], valh.at[s], sems[s][1]).wait()
            pltpu.make_async_copy(
                oc_hbm.at[pl.ds(off, CH)], valc.at[s], sems[s][2]).wait()
            for g in range(CH // L):
                sl = pl.ds(g * L, L)
                idxc[s, sl] = lax.shift_right_logical(idx[s, sl], 5)
            pltpu.sync_copy(valh.at[s], acch.at[idx.at[s]], add=True)
            pltpu.sync_copy(valc.at[s], accc.at[idxc.at[s]], add=True)

        assert nchunk % 2 == 1
        start(0, 0)

        def body(g, carry):
            i0 = 2 * g
            start(i0 + 1, 1)
            finish(i0, 0)
            start(i0 + 2, 0)
            finish(i0 + 1, 1)
            return carry

        lax.fori_loop(0, (nchunk - 1) // 2, body, 0)
        finish(nchunk - 1, 0)
        plsc.subcore_barrier()
        pltpu.sync_copy(acch.at[pl.ds(sid * rpt, rpt)],
                        outh_hbm.at[cid, pl.ds(sid * rpt, rpt)])
        pltpu.sync_copy(accc.at[pl.ds(sid * crpt, crpt)],
                        outc_hbm.at[cid, pl.ds(sid * crpt, crpt)])

    return k(oh, oc, row)


def _node_body(h_ref, c_ref, a0_ref, a1_ref, d0_ref, d1_ref, wa_ref, wb_ref,
               b1_ref, w2_ref, b2_ref, h_out, c_out):
    h = h_ref[:]
    agg = a0_ref[:] + a1_ref[:]
    z = (jnp.dot(h, wa_ref[:], preferred_element_type=F32)
         + jnp.dot(agg, wb_ref[:], preferred_element_type=F32) + b1_ref[:])
    z = _silu(z)
    h_out[:] = jnp.dot(z, w2_ref[:], preferred_element_type=F32) + b2_ref[:]
    c_out[:] = c_ref[:] + d0_ref[:] + d1_ref[:]


def _node_mlp(h, coords, a0, a1, d0, d1, wa, wb, b1, w2, b2):
    n = h.shape[0]
    return pl.pallas_call(
        _node_body,
        grid=(n // NB,),
        in_specs=[
            pl.BlockSpec((NB, HID), lambda i: (i, 0)),
            pl.BlockSpec((NB, 3), lambda i: (i, 0)),
            pl.BlockSpec((NB, HID), lambda i: (i, 0)),
            pl.BlockSpec((NB, HID), lambda i: (i, 0)),
            pl.BlockSpec((NB, 3), lambda i: (i, 0)),
            pl.BlockSpec((NB, 3), lambda i: (i, 0)),
            pl.BlockSpec((HID, HID), lambda i: (0, 0)),
            pl.BlockSpec((HID, HID), lambda i: (0, 0)),
            pl.BlockSpec((1, HID), lambda i: (0, 0)),
            pl.BlockSpec((HID, HID), lambda i: (0, 0)),
            pl.BlockSpec((1, HID), lambda i: (0, 0)),
        ],
        out_specs=[
            pl.BlockSpec((NB, HID), lambda i: (i, 0)),
            pl.BlockSpec((NB, 3), lambda i: (i, 0)),
        ],
        out_shape=[
            jax.ShapeDtypeStruct((n, HID), F32),
            jax.ShapeDtypeStruct((n, 3), F32),
        ],
    )(h, coords, a0, a1, d0, d1, wa, wb, b1, w2, b2)


def kernel(h, coords, edges, mW1, mb1, mW2, mb2, nW1, nb1, nW2, nb2,
           cW1, cb1, cW2, cb2):
    n = h.shape[0]
    e = edges.shape[1]
    row = edges[0].astype(I32)
    col = edges[1].astype(I32)

    lbn = (4.0 * (jnp.arange(n) % PACK)).astype(F32).reshape(n, 1)
    ta, tb = _build_tables(h, coords, lbn, mW1[:HID], mW1[HID:2 * HID],
                           mb1.reshape(1, HID))
    s = _gather_sum(ta, tb, row, col, e)
    oh, oc = _edge_mlp(s, mW2, mb2.reshape(1, HID), cW1,
                       cb1.reshape(1, HID // 2), cW2.reshape(1, HID // 2),
                       cb2.reshape(1, 1), mW1[2 * HID].reshape(1, HID), e)
    acch, accc = _scatter_add(oh, oc, row, n, e)
    d0 = accc[0].reshape(-1, 4)[:n, :3]
    d1 = accc[1].reshape(-1, 4)[:n, :3]
    h_new, coords_new = _node_mlp(h, coords, acch[0][:n], acch[1][:n], d0, d1,
                                  nW1[:HID], nW1[HID:], nb1.reshape(1, HID),
                                  nW2, nb2.reshape(1, HID))
    return (h_new, coords_new)


# speedup vs baseline: 1.0521x; 1.0521x over previous
import functools

import jax
import jax.numpy as jnp
from jax import lax
from jax.experimental import pallas as pl
from jax.experimental.pallas import tpu as pltpu
from jax.experimental.pallas import tpu_sc as plsc

F32 = jnp.float32
I32 = jnp.int32

HID = 128
TW = 256
SW = 144
NC = 2
NS = 16
NW = NC * NS
L = 16

CH = 80
EB = 512
NB = 1000
PACK = 32


def _silu(x):
    return x * jax.nn.sigmoid(x)


def _tables_body(h_ref, c_ref, lb_ref, wa_ref, wb_ref, b1_ref,
                 ta_ref, tb_ref):
    h = h_ref[:]
    c = c_ref[:]
    nb = h.shape[0]
    a = jnp.dot(h, wa_ref[:], preferred_element_type=F32) + b1_ref[:]
    b = jnp.dot(h, wb_ref[:], preferred_element_type=F32)
    ta_ref[:] = jnp.concatenate(
        [a, c, lb_ref[:], jnp.zeros((nb, TW - HID - 4), F32)], axis=1)
    tb_ref[:] = jnp.concatenate(
        [b, -c, jnp.zeros((nb, TW - HID - 3), F32)], axis=1)


def _build_tables(h, coords, lbn, wa, wb, b1):
    n = h.shape[0]
    return pl.pallas_call(
        _tables_body,
        grid=(n // NB,),
        in_specs=[
            pl.BlockSpec((NB, HID), lambda i: (i, 0)),
            pl.BlockSpec((NB, 3), lambda i: (i, 0)),
            pl.BlockSpec((NB, 1), lambda i: (i, 0)),
            pl.BlockSpec((HID, HID), lambda i: (0, 0)),
            pl.BlockSpec((HID, HID), lambda i: (0, 0)),
            pl.BlockSpec((1, HID), lambda i: (0, 0)),
        ],
        out_specs=[
            pl.BlockSpec((NB, TW), lambda i: (i, 0)),
            pl.BlockSpec((NB, TW), lambda i: (i, 0)),
        ],
        out_shape=[
            jax.ShapeDtypeStruct((n, TW), F32),
            jax.ShapeDtypeStruct((n, TW), F32),
        ],
    )(h, coords, lbn, wa, wb, b1)


def _gather_sum(ta, tb, row, col, e):
    epw = e // NW
    nchunk = epw // CH
    mesh = plsc.VectorSubcoreMesh(core_axis_name="c", subcore_axis_name="s")

    @functools.partial(
        pl.kernel,
        out_type=jax.ShapeDtypeStruct((e, SW), F32),
        mesh=mesh,
        scratch_types=[
            pltpu.VMEM((2, CH), I32),
            pltpu.VMEM((2, CH), I32),
            pltpu.VMEM((2, CH, TW), F32),
            pltpu.VMEM((2, CH, TW), F32),
            pltpu.VMEM((CH, SW), F32),
            pltpu.SemaphoreType.DMA,
            pltpu.SemaphoreType.DMA,
            pltpu.SemaphoreType.DMA,
            pltpu.SemaphoreType.DMA,
        ],
    )
    def k(ta_hbm, tb_hbm, row_hbm, col_hbm, out_hbm,
          idxa, idxb, bufa, bufb, sbuf, sa0, sb0, sa1, sb1):
        wid = lax.axis_index("s") * NC + lax.axis_index("c")
        base = wid * epw
        sems = ((sa0, sb0), (sa1, sb1))

        def start(i, s):
            off = base + i * CH
            pltpu.sync_copy(row_hbm.at[pl.ds(off, CH)], idxa.at[s])
            pltpu.sync_copy(col_hbm.at[pl.ds(off, CH)], idxb.at[s])
            pltpu.async_copy(ta_hbm.at[idxa.at[s]], bufa.at[s], sems[s][0])
            pltpu.async_copy(tb_hbm.at[idxb.at[s]], bufb.at[s], sems[s][1])

        def finish(i, s):
            pltpu.make_async_copy(
                ta_hbm.at[idxa.at[s]], bufa.at[s], sems[s][0]).wait()
            pltpu.make_async_copy(
                tb_hbm.at[idxb.at[s]], bufb.at[s], sems[s][1]).wait()

            def add_row(j, c2):
                for t in range(SW // L):
                    sl = pl.ds(t * L, L)
                    sbuf[j, sl] = bufa[s, j, sl] + bufb[s, j, sl]
                return c2

            lax.fori_loop(0, CH, add_row, 0)
            pltpu.sync_copy(sbuf, out_hbm.at[pl.ds(base + i * CH, CH)])

        assert nchunk % 2 == 1
        start(0, 0)

        def body(g, carry):
            i0 = 2 * g
            start(i0 + 1, 1)
            finish(i0, 0)
            start(i0 + 2, 0)
            finish(i0 + 1, 1)
            return carry

        lax.fori_loop(0, (nchunk - 1) // 2, body, 0)
        finish(nchunk - 1, 0)

    return k(ta, tb, row, col)


def _edge_body(s_ref, w2_ref, b2_ref, cw1_ref, cb1_ref, cw2_ref, cb2_ref,
               wr_ref, oh_ref, oc_ref):
    s = s_ref[:]
    pre_h = s[:, :HID]
    d3 = s[:, HID:HID + 3]
    lb = s[:, HID + 3:HID + 4].astype(I32)
    r2 = jnp.sum(d3 * d3, axis=1, keepdims=True)
    r = jnp.sqrt(r2)
    pre = pre_h + r * wr_ref[:]
    act = _silu(pre)
    msg = jnp.dot(act, w2_ref[:], preferred_element_type=F32) + b2_ref[:]
    t = _silu(jnp.dot(msg, cw1_ref[:], preferred_element_type=F32) + cb1_ref[:])
    cw = jnp.sum(t * cw2_ref[:], axis=1, keepdims=True) + cb2_ref[:]
    upd = cw * d3 / (r + 1e-8)
    oh_ref[:] = msg
    lane = lax.broadcasted_iota(I32, (s.shape[0], HID), 1)
    oc = jnp.zeros((s.shape[0], HID), F32)
    for c in range(3):
        oc = oc + jnp.where(lane == lb + c, upd[:, c:c + 1], 0.0)
    oc_ref[:] = oc


def _edge_mlp(s, w2, b2, cw1, cb1, cw2r, cb2, wr, e):
    return pl.pallas_call(
        _edge_body,
        grid=(e // EB,),
        in_specs=[
            pl.BlockSpec((EB, SW), lambda i: (i, 0)),
            pl.BlockSpec((HID, HID), lambda i: (0, 0)),
            pl.BlockSpec((1, HID), lambda i: (0, 0)),
            pl.BlockSpec((HID, HID // 2), lambda i: (0, 0)),
            pl.BlockSpec((1, HID // 2), lambda i: (0, 0)),
            pl.BlockSpec((1, HID // 2), lambda i: (0, 0)),
            pl.BlockSpec((1, 1), lambda i: (0, 0)),
            pl.BlockSpec((1, HID), lambda i: (0, 0)),
        ],
        out_specs=[
            pl.BlockSpec((EB, HID), lambda i: (i, 0)),
            pl.BlockSpec((EB, HID), lambda i: (i, 0)),
        ],
        out_shape=[
            jax.ShapeDtypeStruct((e, HID), F32),
            jax.ShapeDtypeStruct((e, HID), F32),
        ],
    )(s, w2, b2, cw1, cb1, cw2r, cb2, wr)


def _scatter_add(oh, oc, row, n, e):
    epw = e // NW
    nchunk = epw // CH
    npad = (n + 8 * NS - 1) // (8 * NS) * (8 * NS)
    rpt = (npad // NS + 7) // 8 * 8
    npad = rpt * NS
    crpt = ((n + PACK - 1) // PACK + NS - 1) // NS
    crpt = (crpt + 7) // 8 * 8
    ncr = crpt * NS
    zr = 128
    mesh = plsc.VectorSubcoreMesh(core_axis_name="c", subcore_axis_name="s")

    @functools.partial(
        pl.kernel,
        out_type=[
            jax.ShapeDtypeStruct((NC, npad, HID), F32),
            jax.ShapeDtypeStruct((NC, ncr, HID), F32),
        ],
        mesh=mesh,
        scratch_types=[
            pltpu.VMEM_SHARED((npad, HID), F32),
            pltpu.VMEM_SHARED((ncr, HID), F32),
            pltpu.VMEM((2, CH), I32),
            pltpu.VMEM((2, CH), I32),
            pltpu.VMEM((2, CH, HID), F32),
            pltpu.VMEM((2, CH, HID), F32),
            pltpu.SemaphoreType.DMA,
            pltpu.SemaphoreType.DMA,
            pltpu.SemaphoreType.DMA,
            pltpu.SemaphoreType.DMA,
            pltpu.SemaphoreType.DMA,
            pltpu.SemaphoreType.DMA,
        ],
    )
    def k(oh_hbm, oc_hbm, row_hbm, outh_hbm, outc_hbm,
          acch, accc, idx, idxc, valh, valc,
          si0, sh0, sc0, si1, sh1, sc1):
        cid = lax.axis_index("c")
        sid = lax.axis_index("s")
        wid = sid * NC + cid
        base = wid * epw
        sems = ((si0, sh0, sc0), (si1, sh1, sc1))

        def zrow(j, c2):
            for t in range(HID // L):
                valh[0, j, pl.ds(t * L, L)] = jnp.zeros((L,), F32)
            return c2

        lax.fori_loop(0, CH, zrow, 0)
        for kk in range((rpt + CH - 1) // CH):
            sz = min(CH, rpt - kk * CH)
            pltpu.sync_copy(valh.at[0].at[pl.ds(0, sz)],
                            acch.at[pl.ds(sid * rpt + kk * CH, sz)])
        pltpu.sync_copy(valh.at[0].at[pl.ds(0, crpt)],
                        accc.at[pl.ds(sid * crpt, crpt)])
        plsc.subcore_barrier()

        def start(i, s):
            off = base + i * CH
            pltpu.async_copy(row_hbm.at[pl.ds(off, CH)], idx.at[s], sems[s][0])
            pltpu.async_copy(oh_hbm.at[pl.ds(off, CH)], valh.at[s], sems[s][1])
            pltpu.async_copy(oc_hbm.at[pl.ds(off, CH)], valc.at[s], sems[s][2])

        def finish(i, s):
            off = base + i * CH
            pltpu.make_async_copy(
                row_hbm.at[pl.ds(off, CH)], idx.at[s], sems[s][0]).wait()
            pltpu.make_async_copy(
                oh_hbm.at[pl.ds(off, CH)], valh.at[s], sems[s][1]).wait()
            pltpu.make_async_copy(
                oc_hbm.at[pl.ds(off, CH)], valc.at[s], sems[s][2]).wait()
            for g in range(CH // L):
                sl = pl.ds(g * L, L)
                idxc[s, sl] = lax.shift_right_logical(idx[s, sl], 5)
            pltpu.sync_copy(valh.at[s], acch.at[idx.at[s]], add=True)
            pltpu.sync_copy(valc.at[s], accc.at[idxc.at[s]], add=True)

        assert nchunk % 2 == 1
        start(0, 0)

        def body(g, carry):
            i0 = 2 * g
            start(i0 + 1, 1)
            finish(i0, 0)
            start(i0 + 2, 0)
            finish(i0 + 1, 1)
            return carry

        lax.fori_loop(0, (nchunk - 1) // 2, body, 0)
        finish(nchunk - 1, 0)
        plsc.subcore_barrier()
        pltpu.sync_copy(acch.at[pl.ds(sid * rpt, rpt)],
                        outh_hbm.at[cid, pl.ds(sid * rpt, rpt)])
        pltpu.sync_copy(accc.at[pl.ds(sid * crpt, crpt)],
                        outc_hbm.at[cid, pl.ds(sid * crpt, crpt)])

    return k(oh, oc, row)


def _node_body(h_ref, c_ref, a0_ref, a1_ref, d0_ref, d1_ref, wa_ref, wb_ref,
               b1_ref, w2_ref, b2_ref, h_out, c_out):
    h = h_ref[:]
    agg = a0_ref[:] + a1_ref[:]
    z = (jnp.dot(h, wa_ref[:], preferred_element_type=F32)
         + jnp.dot(agg, wb_ref[:], preferred_element_type=F32) + b1_ref[:])
    z = _silu(z)
    h_out[:] = jnp.dot(z, w2_ref[:], preferred_element_type=F32) + b2_ref[:]
    c_out[:] = c_ref[:] + d0_ref[:] + d1_ref[:]


def _node_mlp(h, coords, a0, a1, d0, d1, wa, wb, b1, w2, b2):
    n = h.shape[0]
    return pl.pallas_call(
        _node_body,
        grid=(n // NB,),
        in_specs=[
            pl.BlockSpec((NB, HID), lambda i: (i, 0)),
            pl.BlockSpec((NB, 3), lambda i: (i, 0)),
            pl.BlockSpec((NB, HID), lambda i: (i, 0)),
            pl.BlockSpec((NB, HID), lambda i: (i, 0)),
            pl.BlockSpec((NB, 3), lambda i: (i, 0)),
            pl.BlockSpec((NB, 3), lambda i: (i, 0)),
            pl.BlockSpec((HID, HID), lambda i: (0, 0)),
            pl.BlockSpec((HID, HID), lambda i: (0, 0)),
            pl.BlockSpec((1, HID), lambda i: (0, 0)),
            pl.BlockSpec((HID, HID), lambda i: (0, 0)),
            pl.BlockSpec((1, HID), lambda i: (0, 0)),
        ],
        out_specs=[
            pl.BlockSpec((NB, HID), lambda i: (i, 0)),
            pl.BlockSpec((NB, 3), lambda i: (i, 0)),
        ],
        out_shape=[
            jax.ShapeDtypeStruct((n, HID), F32),
            jax.ShapeDtypeStruct((n, 3), F32),
        ],
    )(h, coords, a0, a1, d0, d1, wa, wb, b1, w2, b2)


def kernel(h, coords, edges, mW1, mb1, mW2, mb2, nW1, nb1, nW2, nb2,
           cW1, cb1, cW2, cb2):
    n = h.shape[0]
    e = edges.shape[1]
    row = edges[0].astype(I32)
    col = edges[1].astype(I32)

    lbn = (4.0 * (jnp.arange(n) % PACK)).astype(F32).reshape(n, 1)
    ta, tb = _build_tables(h, coords, lbn, mW1[:HID], mW1[HID:2 * HID],
                           mb1.reshape(1, HID))
    s = _gather_sum(ta, tb, row, col, e)
    oh, oc = _edge_mlp(s, mW2, mb2.reshape(1, HID), cW1,
                       cb1.reshape(1, HID // 2), cW2.reshape(1, HID // 2),
                       cb2.reshape(1, 1), mW1[2 * HID].reshape(1, HID), e)
    acch, accc = _scatter_add(oh, oc, row, n, e)
    d0 = accc[0].reshape(-1, 4)[:n, :3]
    d1 = accc[1].reshape(-1, 4)[:n, :3]
    h_new, coords_new = _node_mlp(h, coords, acch[0][:n], acch[1][:n], d0, d1,
                                  nW1[:HID], nW1[HID:], nb1.reshape(1, HID),
                                  nW2, nb2.reshape(1, HID))
    return (h_new, coords_new)


# speedup vs baseline: 1.1882x; 1.1294x over previous
import functools

import jax
import jax.numpy as jnp
from jax import lax
from jax.experimental import pallas as pl
from jax.experimental.pallas import tpu as pltpu
from jax.experimental.pallas import tpu_sc as plsc

F32 = jnp.float32
I32 = jnp.int32

HID = 128
TW = 256
SW = 144
NC = 2
NS = 16
NW = NC * NS
L = 16

CH = 80
EB = 512
NB = 1000
PACK = 32


def _silu(x):
    return x * jax.nn.sigmoid(x)


def _tables_body(h_ref, c_ref, lb_ref, wa_ref, wb_ref, b1_ref,
                 ta_ref, tb_ref):
    h = h_ref[:]
    c = c_ref[:]
    nb = h.shape[0]
    a = jnp.dot(h, wa_ref[:], preferred_element_type=F32) + b1_ref[:]
    b = jnp.dot(h, wb_ref[:], preferred_element_type=F32)
    ta_ref[:] = jnp.concatenate(
        [a, c, lb_ref[:], jnp.zeros((nb, TW - HID - 4), F32)], axis=1)
    tb_ref[:] = jnp.concatenate(
        [b, -c, jnp.zeros((nb, TW - HID - 3), F32)], axis=1)


def _build_tables(h, coords, lbn, wa, wb, b1):
    n = h.shape[0]
    return pl.pallas_call(
        _tables_body,
        grid=(n // NB,),
        in_specs=[
            pl.BlockSpec((NB, HID), lambda i: (i, 0)),
            pl.BlockSpec((NB, 3), lambda i: (i, 0)),
            pl.BlockSpec((NB, 1), lambda i: (i, 0)),
            pl.BlockSpec((HID, HID), lambda i: (0, 0)),
            pl.BlockSpec((HID, HID), lambda i: (0, 0)),
            pl.BlockSpec((1, HID), lambda i: (0, 0)),
        ],
        out_specs=[
            pl.BlockSpec((NB, TW), lambda i: (i, 0)),
            pl.BlockSpec((NB, TW), lambda i: (i, 0)),
        ],
        out_shape=[
            jax.ShapeDtypeStruct((n, TW), F32),
            jax.ShapeDtypeStruct((n, TW), F32),
        ],
    )(h, coords, lbn, wa, wb, b1)


def _gather_sum(ta, tb, row, col, e):
    epw = e // NW
    nchunk = epw // CH
    mesh = plsc.VectorSubcoreMesh(core_axis_name="c", subcore_axis_name="s")

    @functools.partial(
        pl.kernel,
        out_type=jax.ShapeDtypeStruct((e, SW), F32),
        mesh=mesh,
        scratch_types=[
            pltpu.VMEM((2, CH), I32),
            pltpu.VMEM((2, CH), I32),
            pltpu.VMEM((2, CH, TW), F32),
            pltpu.VMEM((2, CH, TW), F32),
            pltpu.VMEM((2, CH, SW), F32),
            pltpu.SemaphoreType.DMA,
            pltpu.SemaphoreType.DMA,
            pltpu.SemaphoreType.DMA,
            pltpu.SemaphoreType.DMA,
            pltpu.SemaphoreType.DMA,
            pltpu.SemaphoreType.DMA,
            pltpu.SemaphoreType.DMA,
            pltpu.SemaphoreType.DMA,
        ],
    )
    def k(ta_hbm, tb_hbm, row_hbm, col_hbm, out_hbm,
          idxa, idxb, bufa, bufb, sbuf,
          ia0, ia1, ga0, gb0, ga1, gb1, ss0, ss1):
        wid = lax.axis_index("s") * NC + lax.axis_index("c")
        base = wid * epw
        isems = (ia0, ia1)
        gsems = ((ga0, gb0), (ga1, gb1))
        ssems = (ss0, ss1)

        def idx_start(i, s):
            off = base + i * CH
            pltpu.async_copy(row_hbm.at[pl.ds(off, CH)], idxa.at[s], isems[s])
            pltpu.async_copy(col_hbm.at[pl.ds(off, CH)], idxb.at[s], isems[s])

        def idx_wait(i, s):
            off = base + i * CH
            pltpu.make_async_copy(
                row_hbm.at[pl.ds(off, CH)], idxa.at[s], isems[s]).wait()
            pltpu.make_async_copy(
                col_hbm.at[pl.ds(off, CH)], idxb.at[s], isems[s]).wait()

        def gather_start(s):
            pltpu.async_copy(ta_hbm.at[idxa.at[s]], bufa.at[s], gsems[s][0])
            pltpu.async_copy(tb_hbm.at[idxb.at[s]], bufb.at[s], gsems[s][1])

        def gather_wait(s):
            pltpu.make_async_copy(
                ta_hbm.at[idxa.at[s]], bufa.at[s], gsems[s][0]).wait()
            pltpu.make_async_copy(
                tb_hbm.at[idxb.at[s]], bufb.at[s], gsems[s][1]).wait()

        def store_start(i, s):
            pltpu.async_copy(
                sbuf.at[s], out_hbm.at[pl.ds(base + i * CH, CH)], ssems[s])

        def store_wait(i, s):
            pltpu.make_async_copy(
                sbuf.at[s], out_hbm.at[pl.ds(base + i * CH, CH)],
                ssems[s]).wait()

        def proc(i, s):
            q = 1 - s

            @pl.when(i + 1 < nchunk)
            def _():
                idx_wait(i + 1, q)
                gather_start(q)

            gather_wait(s)

            @pl.when(i + 2 < nchunk)
            def _():
                idx_start(i + 2, s)

            @pl.when(i >= 2)
            def _():
                store_wait(i - 2, s)

            def add_row(j, c2):
                for t in range(SW // L):
                    sl = pl.ds(t * L, L)
                    sbuf[s, j, sl] = bufa[s, j, sl] + bufb[s, j, sl]
                return c2

            lax.fori_loop(0, CH, add_row, 0)
            store_start(i, s)

        assert nchunk % 2 == 1
        idx_start(0, 0)
        idx_start(1, 1)
        idx_wait(0, 0)
        gather_start(0)

        def body(g, carry):
            proc(2 * g, 0)
            proc(2 * g + 1, 1)
            return carry

        lax.fori_loop(0, (nchunk - 1) // 2, body, 0)
        proc(nchunk - 1, 0)
        store_wait(nchunk - 2, 1)
        store_wait(nchunk - 1, 0)

    return k(ta, tb, row, col)


def _edge_body(s_ref, w2_ref, b2_ref, cw1_ref, cb1_ref, cw2_ref, cb2_ref,
               wr_ref, oh_ref, oc_ref):
    s = s_ref[:]
    pre_h = s[:, :HID]
    d3 = s[:, HID:HID + 3]
    lb = s[:, HID + 3:HID + 4].astype(I32)
    r2 = jnp.sum(d3 * d3, axis=1, keepdims=True)
    r = jnp.sqrt(r2)
    pre = pre_h + r * wr_ref[:]
    act = _silu(pre)
    msg = jnp.dot(act, w2_ref[:], preferred_element_type=F32) + b2_ref[:]
    t = _silu(jnp.dot(msg, cw1_ref[:], preferred_element_type=F32) + cb1_ref[:])
    cw = jnp.sum(t * cw2_ref[:], axis=1, keepdims=True) + cb2_ref[:]
    upd = cw * d3 / (r + 1e-8)
    oh_ref[:] = msg
    lane = lax.broadcasted_iota(I32, (s.shape[0], HID), 1)
    oc = jnp.zeros((s.shape[0], HID), F32)
    for c in range(3):
        oc = oc + jnp.where(lane == lb + c, upd[:, c:c + 1], 0.0)
    oc_ref[:] = oc


def _edge_mlp(s, w2, b2, cw1, cb1, cw2r, cb2, wr, e):
    return pl.pallas_call(
        _edge_body,
        grid=(e // EB,),
        in_specs=[
            pl.BlockSpec((EB, SW), lambda i: (i, 0)),
            pl.BlockSpec((HID, HID), lambda i: (0, 0)),
            pl.BlockSpec((1, HID), lambda i: (0, 0)),
            pl.BlockSpec((HID, HID // 2), lambda i: (0, 0)),
            pl.BlockSpec((1, HID // 2), lambda i: (0, 0)),
            pl.BlockSpec((1, HID // 2), lambda i: (0, 0)),
            pl.BlockSpec((1, 1), lambda i: (0, 0)),
            pl.BlockSpec((1, HID), lambda i: (0, 0)),
        ],
        out_specs=[
            pl.BlockSpec((EB, HID), lambda i: (i, 0)),
            pl.BlockSpec((EB, HID), lambda i: (i, 0)),
        ],
        out_shape=[
            jax.ShapeDtypeStruct((e, HID), F32),
            jax.ShapeDtypeStruct((e, HID), F32),
        ],
    )(s, w2, b2, cw1, cb1, cw2r, cb2, wr)


def _scatter_add(oh, oc, row, n, e):
    epw = e // NW
    nchunk = epw // CH
    npad = (n + 8 * NS - 1) // (8 * NS) * (8 * NS)
    rpt = (npad // NS + 7) // 8 * 8
    npad = rpt * NS
    crpt = ((n + PACK - 1) // PACK + NS - 1) // NS
    crpt = (crpt + 7) // 8 * 8
    ncr = crpt * NS
    zr = 128
    mesh = plsc.VectorSubcoreMesh(core_axis_name="c", subcore_axis_name="s")

    @functools.partial(
        pl.kernel,
        out_type=[
            jax.ShapeDtypeStruct((NC, npad, HID), F32),
            jax.ShapeDtypeStruct((NC, ncr, HID), F32),
        ],
        mesh=mesh,
        scratch_types=[
            pltpu.VMEM_SHARED((npad, HID), F32),
            pltpu.VMEM_SHARED((ncr, HID), F32),
            pltpu.VMEM((2, CH), I32),
            pltpu.VMEM((2, CH), I32),
            pltpu.VMEM((2, CH, HID), F32),
            pltpu.VMEM((2, CH, HID), F32),
            pltpu.SemaphoreType.DMA,
            pltpu.SemaphoreType.DMA,
            pltpu.SemaphoreType.DMA,
            pltpu.SemaphoreType.DMA,
            pltpu.SemaphoreType.DMA,
            pltpu.SemaphoreType.DMA,
        ],
    )
    def k(oh_hbm, oc_hbm, row_hbm, outh_hbm, outc_hbm,
          acch, accc, idx, idxc, valh, valc,
          si0, sh0, sc0, si1, sh1, sc1):
        cid = lax.axis_index("c")
        sid = lax.axis_index("s")
        wid = sid * NC + cid
        base = wid * epw
        sems = ((si0, sh0, sc0), (si1, sh1, sc1))

        def zrow(j, c2):
            for t in range(HID // L):
                valh[0, j, pl.ds(t * L, L)] = jnp.zeros((L,), F32)
            return c2

        lax.fori_loop(0, CH, zrow, 0)
        for kk in range((rpt + CH - 1) // CH):
            sz = min(CH, rpt - kk * CH)
            pltpu.sync_copy(valh.at[0].at[pl.ds(0, sz)],
                            acch.at[pl.ds(sid * rpt + kk * CH, sz)])
        pltpu.sync_copy(valh.at[0].at[pl.ds(0, crpt)],
                        accc.at[pl.ds(sid * crpt, crpt)])
        plsc.subcore_barrier()

        def start(i, s):
            off = base + i * CH
            pltpu.async_copy(row_hbm.at[pl.ds(off, CH)], idx.at[s], sems[s][0])
            pltpu.async_copy(oh_hbm.at[pl.ds(off, CH)], valh.at[s], sems[s][1])
            pltpu.async_copy(oc_hbm.at[pl.ds(off, CH)], valc.at[s], sems[s][2])

        def finish(i, s):
            off = base + i * CH
            pltpu.make_async_copy(
                row_hbm.at[pl.ds(off, CH)], idx.at[s], sems[s][0]).wait()
            pltpu.make_async_copy(
                oh_hbm.at[pl.ds(off, CH)], valh.at[s], sems[s][1]).wait()
            pltpu.make_async_copy(
                oc_hbm.at[pl.ds(off, CH)], valc.at[s], sems[s][2]).wait()
            for g in range(CH // L):
                sl = pl.ds(g * L, L)
                idxc[s, sl] = lax.shift_right_logical(idx[s, sl], 5)
            pltpu.sync_copy(valh.at[s], acch.at[idx.at[s]], add=True)
            pltpu.sync_copy(valc.at[s], accc.at[idxc.at[s]], add=True)

        assert nchunk % 2 == 1
        start(0, 0)

        def body(g, carry):
            i0 = 2 * g
            start(i0 + 1, 1)
            finish(i0, 0)
            start(i0 + 2, 0)
            finish(i0 + 1, 1)
            return carry

        lax.fori_loop(0, (nchunk - 1) // 2, body, 0)
        finish(nchunk - 1, 0)
        plsc.subcore_barrier()
        pltpu.sync_copy(acch.at[pl.ds(sid * rpt, rpt)],
                        outh_hbm.at[cid, pl.ds(sid * rpt, rpt)])
        pltpu.sync_copy(accc.at[pl.ds(sid * crpt, crpt)],
                        outc_hbm.at[cid, pl.ds(sid * crpt, crpt)])

    return k(oh, oc, row)


def _node_body(h_ref, c_ref, a0_ref, a1_ref, d0_ref, d1_ref, wa_ref, wb_ref,
               b1_ref, w2_ref, b2_ref, h_out, c_out):
    h = h_ref[:]
    agg = a0_ref[:] + a1_ref[:]
    z = (jnp.dot(h, wa_ref[:], preferred_element_type=F32)
         + jnp.dot(agg, wb_ref[:], preferred_element_type=F32) + b1_ref[:])
    z = _silu(z)
    h_out[:] = jnp.dot(z, w2_ref[:], preferred_element_type=F32) + b2_ref[:]
    c_out[:] = c_ref[:] + d0_ref[:] + d1_ref[:]


def _node_mlp(h, coords, a0, a1, d0, d1, wa, wb, b1, w2, b2):
    n = h.shape[0]
    return pl.pallas_call(
        _node_body,
        grid=(n // NB,),
        in_specs=[
            pl.BlockSpec((NB, HID), lambda i: (i, 0)),
            pl.BlockSpec((NB, 3), lambda i: (i, 0)),
            pl.BlockSpec((NB, HID), lambda i: (i, 0)),
            pl.BlockSpec((NB, HID), lambda i: (i, 0)),
            pl.BlockSpec((NB, 3), lambda i: (i, 0)),
            pl.BlockSpec((NB, 3), lambda i: (i, 0)),
            pl.BlockSpec((HID, HID), lambda i: (0, 0)),
            pl.BlockSpec((HID, HID), lambda i: (0, 0)),
            pl.BlockSpec((1, HID), lambda i: (0, 0)),
            pl.BlockSpec((HID, HID), lambda i: (0, 0)),
            pl.BlockSpec((1, HID), lambda i: (0, 0)),
        ],
        out_specs=[
            pl.BlockSpec((NB, HID), lambda i: (i, 0)),
            pl.BlockSpec((NB, 3), lambda i: (i, 0)),
        ],
        out_shape=[
            jax.ShapeDtypeStruct((n, HID), F32),
            jax.ShapeDtypeStruct((n, 3), F32),
        ],
    )(h, coords, a0, a1, d0, d1, wa, wb, b1, w2, b2)


def kernel(h, coords, edges, mW1, mb1, mW2, mb2, nW1, nb1, nW2, nb2,
           cW1, cb1, cW2, cb2):
    n = h.shape[0]
    e = edges.shape[1]
    row = edges[0].astype(I32)
    col = edges[1].astype(I32)

    lbn = (4.0 * (jnp.arange(n) % PACK)).astype(F32).reshape(n, 1)
    ta, tb = _build_tables(h, coords, lbn, mW1[:HID], mW1[HID:2 * HID],
                           mb1.reshape(1, HID))
    s = _gather_sum(ta, tb, row, col, e)
    oh, oc = _edge_mlp(s, mW2, mb2.reshape(1, HID), cW1,
                       cb1.reshape(1, HID // 2), cW2.reshape(1, HID // 2),
                       cb2.reshape(1, 1), mW1[2 * HID].reshape(1, HID), e)
    acch, accc = _scatter_add(oh, oc, row, n, e)
    d0 = accc[0].reshape(-1, 4)[:n, :3]
    d1 = accc[1].reshape(-1, 4)[:n, :3]
    h_new, coords_new = _node_mlp(h, coords, acch[0][:n], acch[1][:n], d0, d1,
                                  nW1[:HID], nW1[HID:], nb1.reshape(1, HID),
                                  nW2, nb2.reshape(1, HID))
    return (h_new, coords_new)


# speedup vs baseline: 1.1883x; 1.0001x over previous
import functools

import jax
import jax.numpy as jnp
from jax import lax
from jax.experimental import pallas as pl
from jax.experimental.pallas import tpu as pltpu
from jax.experimental.pallas import tpu_sc as plsc

F32 = jnp.float32
I32 = jnp.int32

HID = 128
TW = 256
SW = 144
NC = 2
NS = 16
NW = NC * NS
L = 16

NSLAB = 1
CH = 80
EB = 512
NB = 1000
PACK = 32


def _silu(x):
    return x * jax.nn.sigmoid(x)


def _tables_body(h_ref, c_ref, lb_ref, wa_ref, wb_ref, b1_ref,
                 ta_ref, tb_ref):
    h = h_ref[:]
    c = c_ref[:]
    nb = h.shape[0]
    a = jnp.dot(h, wa_ref[:], preferred_element_type=F32) + b1_ref[:]
    b = jnp.dot(h, wb_ref[:], preferred_element_type=F32)
    ta_ref[:] = jnp.concatenate(
        [a, c, lb_ref[:], jnp.zeros((nb, TW - HID - 4), F32)], axis=1)
    tb_ref[:] = jnp.concatenate(
        [b, -c, jnp.zeros((nb, TW - HID - 3), F32)], axis=1)


def _build_tables(h, coords, lbn, wa, wb, b1):
    n = h.shape[0]
    return pl.pallas_call(
        _tables_body,
        grid=(n // NB,),
        in_specs=[
            pl.BlockSpec((NB, HID), lambda i: (i, 0)),
            pl.BlockSpec((NB, 3), lambda i: (i, 0)),
            pl.BlockSpec((NB, 1), lambda i: (i, 0)),
            pl.BlockSpec((HID, HID), lambda i: (0, 0)),
            pl.BlockSpec((HID, HID), lambda i: (0, 0)),
            pl.BlockSpec((1, HID), lambda i: (0, 0)),
        ],
        out_specs=[
            pl.BlockSpec((NB, TW), lambda i: (i, 0)),
            pl.BlockSpec((NB, TW), lambda i: (i, 0)),
        ],
        out_shape=[
            jax.ShapeDtypeStruct((n, TW), F32),
            jax.ShapeDtypeStruct((n, TW), F32),
        ],
    )(h, coords, lbn, wa, wb, b1)


def _gather_sum(ta, tb, row, col, e):
    epw = e // NW
    nchunk = epw // CH
    mesh = plsc.VectorSubcoreMesh(core_axis_name="c", subcore_axis_name="s")

    @functools.partial(
        pl.kernel,
        out_type=jax.ShapeDtypeStruct((e, SW), F32),
        mesh=mesh,
        scratch_types=[
            pltpu.VMEM((2, CH), I32),
            pltpu.VMEM((2, CH), I32),
            pltpu.VMEM((2, CH, TW), F32),
            pltpu.VMEM((2, CH, TW), F32),
            pltpu.VMEM((2, CH, SW), F32),
            pltpu.SemaphoreType.DMA,
            pltpu.SemaphoreType.DMA,
            pltpu.SemaphoreType.DMA,
            pltpu.SemaphoreType.DMA,
            pltpu.SemaphoreType.DMA,
            pltpu.SemaphoreType.DMA,
            pltpu.SemaphoreType.DMA,
            pltpu.SemaphoreType.DMA,
        ],
    )
    def k(ta_hbm, tb_hbm, row_hbm, col_hbm, out_hbm,
          idxa, idxb, bufa, bufb, sbuf,
          ia0, ia1, ga0, gb0, ga1, gb1, ss0, ss1):
        wid = lax.axis_index("s") * NC + lax.axis_index("c")
        base = wid * epw
        isems = (ia0, ia1)
        gsems = ((ga0, gb0), (ga1, gb1))
        ssems = (ss0, ss1)

        def idx_start(i, s):
            off = base + i * CH
            pltpu.async_copy(row_hbm.at[pl.ds(off, CH)], idxa.at[s], isems[s])
            pltpu.async_copy(col_hbm.at[pl.ds(off, CH)], idxb.at[s], isems[s])

        def idx_wait(i, s):
            off = base + i * CH
            pltpu.make_async_copy(
                row_hbm.at[pl.ds(off, CH)], idxa.at[s], isems[s]).wait()
            pltpu.make_async_copy(
                col_hbm.at[pl.ds(off, CH)], idxb.at[s], isems[s]).wait()

        def gather_start(s):
            pltpu.async_copy(ta_hbm.at[idxa.at[s]], bufa.at[s], gsems[s][0])
            pltpu.async_copy(tb_hbm.at[idxb.at[s]], bufb.at[s], gsems[s][1])

        def gather_wait(s):
            pltpu.make_async_copy(
                ta_hbm.at[idxa.at[s]], bufa.at[s], gsems[s][0]).wait()
            pltpu.make_async_copy(
                tb_hbm.at[idxb.at[s]], bufb.at[s], gsems[s][1]).wait()

        def store_start(i, s):
            pltpu.async_copy(
                sbuf.at[s], out_hbm.at[pl.ds(base + i * CH, CH)], ssems[s])

        def store_wait(i, s):
            pltpu.make_async_copy(
                sbuf.at[s], out_hbm.at[pl.ds(base + i * CH, CH)],
                ssems[s]).wait()

        def proc(i, s):
            q = 1 - s

            @pl.when(i + 1 < nchunk)
            def _():
                idx_wait(i + 1, q)
                gather_start(q)

            gather_wait(s)

            @pl.when(i + 2 < nchunk)
            def _():
                idx_start(i + 2, s)

            @pl.when(i >= 2)
            def _():
                store_wait(i - 2, s)

            def add_row(j, c2):
                for t in range(SW // L):
                    sl = pl.ds(t * L, L)
                    sbuf[s, j, sl] = bufa[s, j, sl] + bufb[s, j, sl]
                return c2

            lax.fori_loop(0, CH, add_row, 0)
            store_start(i, s)

        assert nchunk % 2 == 1
        idx_start(0, 0)
        idx_start(1, 1)
        idx_wait(0, 0)
        gather_start(0)

        def body(g, carry):
            proc(2 * g, 0)
            proc(2 * g + 1, 1)
            return carry

        lax.fori_loop(0, (nchunk - 1) // 2, body, 0)
        proc(nchunk - 1, 0)
        store_wait(nchunk - 2, 1)
        store_wait(nchunk - 1, 0)

    return k(ta, tb, row, col)


def _edge_body(s_ref, w2_ref, b2_ref, cw1_ref, cb1_ref, cw2_ref, cb2_ref,
               wr_ref, oh_ref, oc_ref):
    s = s_ref[:]
    pre_h = s[:, :HID]
    d3 = s[:, HID:HID + 3]
    lb = s[:, HID + 3:HID + 4].astype(I32)
    r2 = jnp.sum(d3 * d3, axis=1, keepdims=True)
    r = jnp.sqrt(r2)
    pre = pre_h + r * wr_ref[:]
    act = _silu(pre)
    msg = jnp.dot(act.astype(jnp.bfloat16), w2_ref[:].astype(jnp.bfloat16),
                  preferred_element_type=F32) + b2_ref[:]
    t = _silu(jnp.dot(msg.astype(jnp.bfloat16),
                      cw1_ref[:].astype(jnp.bfloat16),
                      preferred_element_type=F32) + cb1_ref[:])
    cw = jnp.sum(t * cw2_ref[:], axis=1, keepdims=True) + cb2_ref[:]
    upd = cw * d3 / (r + 1e-8)
    oh_ref[:] = msg
    lane = lax.broadcasted_iota(I32, (s.shape[0], HID), 1)
    oc = jnp.zeros((s.shape[0], HID), F32)
    for c in range(3):
        oc = oc + jnp.where(lane == lb + c, upd[:, c:c + 1], 0.0)
    oc_ref[:] = oc


def _edge_mlp(s, w2, b2, cw1, cb1, cw2r, cb2, wr, e):
    return pl.pallas_call(
        _edge_body,
        grid=(e // EB,),
        in_specs=[
            pl.BlockSpec((EB, SW), lambda i: (i, 0)),
            pl.BlockSpec((HID, HID), lambda i: (0, 0)),
            pl.BlockSpec((1, HID), lambda i: (0, 0)),
            pl.BlockSpec((HID, HID // 2), lambda i: (0, 0)),
            pl.BlockSpec((1, HID // 2), lambda i: (0, 0)),
            pl.BlockSpec((1, HID // 2), lambda i: (0, 0)),
            pl.BlockSpec((1, 1), lambda i: (0, 0)),
            pl.BlockSpec((1, HID), lambda i: (0, 0)),
        ],
        out_specs=[
            pl.BlockSpec((EB, HID), lambda i: (i, 0)),
            pl.BlockSpec((EB, HID), lambda i: (i, 0)),
        ],
        out_shape=[
            jax.ShapeDtypeStruct((e, HID), F32),
            jax.ShapeDtypeStruct((e, HID), F32),
        ],
    )(s, w2, b2, cw1, cb1, cw2r, cb2, wr)


def _scatter_add(oh, oc, row, n, e):
    epw = e // NW
    nchunk = epw // CH
    npad = (n + 8 * NS - 1) // (8 * NS) * (8 * NS)
    rpt = (npad // NS + 7) // 8 * 8
    npad = rpt * NS
    crpt = ((n + PACK - 1) // PACK + NS - 1) // NS
    crpt = (crpt + 7) // 8 * 8
    ncr = crpt * NS
    zr = 128
    mesh = plsc.VectorSubcoreMesh(core_axis_name="c", subcore_axis_name="s")

    @functools.partial(
        pl.kernel,
        out_type=[
            jax.ShapeDtypeStruct((NC, npad, HID), F32),
            jax.ShapeDtypeStruct((NC, ncr, HID), F32),
        ],
        mesh=mesh,
        scratch_types=[
            pltpu.VMEM_SHARED((npad, HID), F32),
            pltpu.VMEM_SHARED((ncr, HID), F32),
            pltpu.VMEM((2, CH), I32),
            pltpu.VMEM((2, CH), I32),
            pltpu.VMEM((2, CH, HID), F32),
            pltpu.VMEM((2, CH, HID), F32),
            pltpu.SemaphoreType.DMA,
            pltpu.SemaphoreType.DMA,
            pltpu.SemaphoreType.DMA,
            pltpu.SemaphoreType.DMA,
            pltpu.SemaphoreType.DMA,
            pltpu.SemaphoreType.DMA,
        ],
    )
    def k(oh_hbm, oc_hbm, row_hbm, outh_hbm, outc_hbm,
          acch, accc, idx, idxc, valh, valc,
          si0, sh0, sc0, si1, sh1, sc1):
        cid = lax.axis_index("c")
        sid = lax.axis_index("s")
        wid = sid * NC + cid
        base = wid * epw
        sems = ((si0, sh0, sc0), (si1, sh1, sc1))

        def zrow(j, c2):
            for t in range(HID // L):
                valh[0, j, pl.ds(t * L, L)] = jnp.zeros((L,), F32)
            return c2

        lax.fori_loop(0, CH, zrow, 0)
        for kk in range((rpt + CH - 1) // CH):
            sz = min(CH, rpt - kk * CH)
            pltpu.sync_copy(valh.at[0].at[pl.ds(0, sz)],
                            acch.at[pl.ds(sid * rpt + kk * CH, sz)])
        pltpu.sync_copy(valh.at[0].at[pl.ds(0, crpt)],
                        accc.at[pl.ds(sid * crpt, crpt)])
        plsc.subcore_barrier()

        def start(i, s):
            off = base + i * CH
            pltpu.async_copy(row_hbm.at[pl.ds(off, CH)], idx.at[s], sems[s][0])
            pltpu.async_copy(oh_hbm.at[pl.ds(off, CH)], valh.at[s], sems[s][1])
            pltpu.async_copy(oc_hbm.at[pl.ds(off, CH)], valc.at[s], sems[s][2])

        def finish(i, s):
            off = base + i * CH
            pltpu.make_async_copy(
                row_hbm.at[pl.ds(off, CH)], idx.at[s], sems[s][0]).wait()
            pltpu.make_async_copy(
                oh_hbm.at[pl.ds(off, CH)], valh.at[s], sems[s][1]).wait()
            pltpu.make_async_copy(
                oc_hbm.at[pl.ds(off, CH)], valc.at[s], sems[s][2]).wait()
            for g in range(CH // L):
                sl = pl.ds(g * L, L)
                idxc[s, sl] = lax.shift_right_logical(idx[s, sl], 5)
            pltpu.sync_copy(valh.at[s], acch.at[idx.at[s]], add=True)
            pltpu.sync_copy(valc.at[s], accc.at[idxc.at[s]], add=True)

        assert nchunk % 2 == 1
        start(0, 0)

        def body(g, carry):
            i0 = 2 * g
            start(i0 + 1, 1)
            finish(i0, 0)
            start(i0 + 2, 0)
            finish(i0 + 1, 1)
            return carry

        lax.fori_loop(0, (nchunk - 1) // 2, body, 0)
        finish(nchunk - 1, 0)
        plsc.subcore_barrier()
        pltpu.sync_copy(acch.at[pl.ds(sid * rpt, rpt)],
                        outh_hbm.at[cid, pl.ds(sid * rpt, rpt)])
        pltpu.sync_copy(accc.at[pl.ds(sid * crpt, crpt)],
                        outc_hbm.at[cid, pl.ds(sid * crpt, crpt)])

    return k(oh, oc, row)


def _node_mlp(h, coords, accs, deltas, wa, wb, b1, w2, b2):
    n = h.shape[0]
    na = len(accs)
    nd = len(deltas)

    def body(*refs):
        h_ref, c_ref = refs[0], refs[1]
        a_refs = refs[2:2 + na]
        d_refs = refs[2 + na:2 + na + nd]
        wa_ref, wb_ref, b1_ref, w2_ref, b2_ref = refs[2 + na + nd:-2]
        h_out, c_out = refs[-2], refs[-1]
        h_ = h_ref[:]
        agg = a_refs[0][:]
        for a in a_refs[1:]:
            agg = agg + a[:]
        z = (jnp.dot(h_, wa_ref[:], preferred_element_type=F32)
             + jnp.dot(agg, wb_ref[:], preferred_element_type=F32)
             + b1_ref[:])
        z = _silu(z)
        h_out[:] = jnp.dot(z, w2_ref[:], preferred_element_type=F32) + b2_ref[:]
        cc = c_ref[:]
        for d in d_refs:
            cc = cc + d[:]
        c_out[:] = cc

    return pl.pallas_call(
        body,
        grid=(n // NB,),
        in_specs=(
            [pl.BlockSpec((NB, HID), lambda i: (i, 0)),
             pl.BlockSpec((NB, 3), lambda i: (i, 0))]
            + [pl.BlockSpec((NB, HID), lambda i: (i, 0))] * na
            + [pl.BlockSpec((NB, 3), lambda i: (i, 0))] * nd
            + [pl.BlockSpec((HID, HID), lambda i: (0, 0)),
               pl.BlockSpec((HID, HID), lambda i: (0, 0)),
               pl.BlockSpec((1, HID), lambda i: (0, 0)),
               pl.BlockSpec((HID, HID), lambda i: (0, 0)),
               pl.BlockSpec((1, HID), lambda i: (0, 0))]
        ),
        out_specs=[
            pl.BlockSpec((NB, HID), lambda i: (i, 0)),
            pl.BlockSpec((NB, 3), lambda i: (i, 0)),
        ],
        out_shape=[
            jax.ShapeDtypeStruct((n, HID), F32),
            jax.ShapeDtypeStruct((n, 3), F32),
        ],
    )(h, coords, *accs, *deltas, wa, wb, b1, w2, b2)


def kernel(h, coords, edges, mW1, mb1, mW2, mb2, nW1, nb1, nW2, nb2,
           cW1, cb1, cW2, cb2):
    n = h.shape[0]
    e = edges.shape[1]
    row = edges[0].astype(I32)
    col = edges[1].astype(I32)

    lbn = (4.0 * (jnp.arange(n) % PACK)).astype(F32).reshape(n, 1)
    ta, tb = _build_tables(h, coords, lbn, mW1[:HID], mW1[HID:2 * HID],
                           mb1.reshape(1, HID))
    es = e // NSLAB
    accs, deltas = [], []
    for k in range(NSLAB):
        rk = lax.dynamic_slice_in_dim(row, k * es, es)
        ck = lax.dynamic_slice_in_dim(col, k * es, es)
        s = _gather_sum(ta, tb, rk, ck, es)
        oh, oc = _edge_mlp(s, mW2, mb2.reshape(1, HID), cW1,
                           cb1.reshape(1, HID // 2), cW2.reshape(1, HID // 2),
                           cb2.reshape(1, 1), mW1[2 * HID].reshape(1, HID), es)
        acch, accc = _scatter_add(oh, oc, rk, n, es)
        accs += [acch[0][:n], acch[1][:n]]
        deltas += [accc[0].reshape(-1, 4)[:n, :3],
                   accc[1].reshape(-1, 4)[:n, :3]]
    h_new, coords_new = _node_mlp(h, coords, accs, deltas,
                                  nW1[:HID], nW1[HID:], nb1.reshape(1, HID),
                                  nW2, nb2.reshape(1, HID))
    return (h_new, coords_new)


# speedup vs baseline: 1.1885x; 1.0001x over previous
import functools

import jax
import jax.numpy as jnp
from jax import lax
from jax.experimental import pallas as pl
from jax.experimental.pallas import tpu as pltpu
from jax.experimental.pallas import tpu_sc as plsc

F32 = jnp.float32
I32 = jnp.int32

HID = 128
TW = 256
SW = 144
NC = 2
NS = 16
NW = NC * NS
L = 16

NSLAB = 1
CH = 80
EB = 512
NB = 1000
PACK = 32


def _silu(x):
    return x * jax.nn.sigmoid(x)


def _tables_body(h_ref, c_ref, lb_ref, wa_ref, wb_ref, b1_ref,
                 ta_ref, tb_ref):
    h = h_ref[:]
    c = c_ref[:]
    nb = h.shape[0]
    a = jnp.dot(h, wa_ref[:], preferred_element_type=F32) + b1_ref[:]
    b = jnp.dot(h, wb_ref[:], preferred_element_type=F32)
    ta_ref[:] = jnp.concatenate(
        [a, c, lb_ref[:], jnp.zeros((nb, TW - HID - 4), F32)], axis=1)
    tb_ref[:] = jnp.concatenate(
        [b, -c, jnp.zeros((nb, TW - HID - 3), F32)], axis=1)


def _build_tables(h, coords, lbn, wa, wb, b1):
    n = h.shape[0]
    return pl.pallas_call(
        _tables_body,
        grid=(n // NB,),
        in_specs=[
            pl.BlockSpec((NB, HID), lambda i: (i, 0)),
            pl.BlockSpec((NB, 3), lambda i: (i, 0)),
            pl.BlockSpec((NB, 1), lambda i: (i, 0)),
            pl.BlockSpec((HID, HID), lambda i: (0, 0)),
            pl.BlockSpec((HID, HID), lambda i: (0, 0)),
            pl.BlockSpec((1, HID), lambda i: (0, 0)),
        ],
        out_specs=[
            pl.BlockSpec((NB, TW), lambda i: (i, 0)),
            pl.BlockSpec((NB, TW), lambda i: (i, 0)),
        ],
        out_shape=[
            jax.ShapeDtypeStruct((n, TW), F32),
            jax.ShapeDtypeStruct((n, TW), F32),
        ],
    )(h, coords, lbn, wa, wb, b1)


def _gather_sum(ta, tb, row, col, e):
    epw = e // NW
    nchunk = epw // CH
    mesh = plsc.VectorSubcoreMesh(core_axis_name="c", subcore_axis_name="s")

    @functools.partial(
        pl.kernel,
        out_type=jax.ShapeDtypeStruct((e, SW), F32),
        mesh=mesh,
        scratch_types=[
            pltpu.VMEM((2, CH), I32),
            pltpu.VMEM((2, CH), I32),
            pltpu.VMEM((2, CH, TW), F32),
            pltpu.VMEM((2, CH, TW), F32),
            pltpu.VMEM((2, CH, SW), F32),
            pltpu.SemaphoreType.DMA,
            pltpu.SemaphoreType.DMA,
            pltpu.SemaphoreType.DMA,
            pltpu.SemaphoreType.DMA,
            pltpu.SemaphoreType.DMA,
            pltpu.SemaphoreType.DMA,
            pltpu.SemaphoreType.DMA,
            pltpu.SemaphoreType.DMA,
        ],
    )
    def k(ta_hbm, tb_hbm, row_hbm, col_hbm, out_hbm,
          idxa, idxb, bufa, bufb, sbuf,
          ia0, ia1, ga0, gb0, ga1, gb1, ss0, ss1):
        wid = lax.axis_index("s") * NC + lax.axis_index("c")
        base = wid * epw
        isems = (ia0, ia1)
        gsems = ((ga0, gb0), (ga1, gb1))
        ssems = (ss0, ss1)

        def idx_start(i, s):
            off = base + i * CH
            pltpu.async_copy(row_hbm.at[pl.ds(off, CH)], idxa.at[s], isems[s])
            pltpu.async_copy(col_hbm.at[pl.ds(off, CH)], idxb.at[s], isems[s])

        def idx_wait(i, s):
            off = base + i * CH
            pltpu.make_async_copy(
                row_hbm.at[pl.ds(off, CH)], idxa.at[s], isems[s]).wait()
            pltpu.make_async_copy(
                col_hbm.at[pl.ds(off, CH)], idxb.at[s], isems[s]).wait()

        def gather_start(s):
            pltpu.async_copy(ta_hbm.at[idxa.at[s]], bufa.at[s], gsems[s][0])
            pltpu.async_copy(tb_hbm.at[idxb.at[s]], bufb.at[s], gsems[s][1])

        def gather_wait(s):
            pltpu.make_async_copy(
                ta_hbm.at[idxa.at[s]], bufa.at[s], gsems[s][0]).wait()
            pltpu.make_async_copy(
                tb_hbm.at[idxb.at[s]], bufb.at[s], gsems[s][1]).wait()

        def store_start(i, s):
            pltpu.async_copy(
                sbuf.at[s], out_hbm.at[pl.ds(base + i * CH, CH)], ssems[s])

        def store_wait(i, s):
            pltpu.make_async_copy(
                sbuf.at[s], out_hbm.at[pl.ds(base + i * CH, CH)],
                ssems[s]).wait()

        def proc(i, s):
            q = 1 - s

            @pl.when(i + 1 < nchunk)
            def _():
                idx_wait(i + 1, q)
                gather_start(q)

            gather_wait(s)

            @pl.when(i + 2 < nchunk)
            def _():
                idx_start(i + 2, s)

            @pl.when(i >= 2)
            def _():
                store_wait(i - 2, s)

            def add_row(j, c2):
                for t in range(SW // L):
                    sl = pl.ds(t * L, L)
                    sbuf[s, j, sl] = bufa[s, j, sl] + bufb[s, j, sl]
                return c2

            lax.fori_loop(0, CH, add_row, 0)
            store_start(i, s)

        assert nchunk % 2 == 1
        idx_start(0, 0)
        idx_start(1, 1)
        idx_wait(0, 0)
        gather_start(0)

        def body(g, carry):
            proc(2 * g, 0)
            proc(2 * g + 1, 1)
            return carry

        lax.fori_loop(0, (nchunk - 1) // 2, body, 0)
        proc(nchunk - 1, 0)
        store_wait(nchunk - 2, 1)
        store_wait(nchunk - 1, 0)

    return k(ta, tb, row, col)


def _edge_body(s_ref, w2_ref, b2_ref, cw1_ref, cb1_ref, cw2_ref, cb2_ref,
               wr_ref, oh_ref, oc_ref):
    s = s_ref[:]
    pre_h = s[:, :HID]
    d3 = s[:, HID:HID + 3]
    lb = s[:, HID + 3:HID + 4].astype(I32)
    r2 = jnp.sum(d3 * d3, axis=1, keepdims=True)
    r = jnp.sqrt(r2)
    pre = pre_h + r * wr_ref[:]
    act = _silu(pre)
    msg = jnp.dot(act, w2_ref[:], preferred_element_type=F32) + b2_ref[:]
    t = _silu(jnp.dot(msg, cw1_ref[:], preferred_element_type=F32) + cb1_ref[:])
    cw = jnp.sum(t * cw2_ref[:], axis=1, keepdims=True) + cb2_ref[:]
    upd = cw * d3 / (r + 1e-8)
    oh_ref[:] = msg
    lane = lax.broadcasted_iota(I32, (s.shape[0], HID), 1)
    oc = jnp.zeros((s.shape[0], HID), F32)
    for c in range(3):
        oc = oc + jnp.where(lane == lb + c, upd[:, c:c + 1], 0.0)
    oc_ref[:] = oc


def _edge_mlp(s, w2, b2, cw1, cb1, cw2r, cb2, wr, e):
    return pl.pallas_call(
        _edge_body,
        grid=(e // EB,),
        in_specs=[
            pl.BlockSpec((EB, SW), lambda i: (i, 0)),
            pl.BlockSpec((HID, HID), lambda i: (0, 0)),
            pl.BlockSpec((1, HID), lambda i: (0, 0)),
            pl.BlockSpec((HID, HID // 2), lambda i: (0, 0)),
            pl.BlockSpec((1, HID // 2), lambda i: (0, 0)),
            pl.BlockSpec((1, HID // 2), lambda i: (0, 0)),
            pl.BlockSpec((1, 1), lambda i: (0, 0)),
            pl.BlockSpec((1, HID), lambda i: (0, 0)),
        ],
        out_specs=[
            pl.BlockSpec((EB, HID), lambda i: (i, 0)),
            pl.BlockSpec((EB, HID), lambda i: (i, 0)),
        ],
        out_shape=[
            jax.ShapeDtypeStruct((e, HID), F32),
            jax.ShapeDtypeStruct((e, HID), F32),
        ],
    )(s, w2, b2, cw1, cb1, cw2r, cb2, wr)


def _scatter_add(oh, oc, row, n, e):
    epw = e // NW
    nchunk = epw // CH
    npad = (n + 8 * NS - 1) // (8 * NS) * (8 * NS)
    rpt = (npad // NS + 7) // 8 * 8
    npad = rpt * NS
    crpt = ((n + PACK - 1) // PACK + NS - 1) // NS
    crpt = (crpt + 7) // 8 * 8
    ncr = crpt * NS
    zr = 128
    mesh = plsc.VectorSubcoreMesh(core_axis_name="c", subcore_axis_name="s")

    @functools.partial(
        pl.kernel,
        out_type=[
            jax.ShapeDtypeStruct((NC, npad, HID), F32),
            jax.ShapeDtypeStruct((NC, ncr, HID), F32),
        ],
        mesh=mesh,
        scratch_types=[
            pltpu.VMEM_SHARED((npad, HID), F32),
            pltpu.VMEM_SHARED((ncr, HID), F32),
            pltpu.VMEM((2, CH), I32),
            pltpu.VMEM((2, CH), I32),
            pltpu.VMEM((2, CH, HID), F32),
            pltpu.VMEM((2, CH, HID), F32),
            pltpu.SemaphoreType.DMA,
            pltpu.SemaphoreType.DMA,
            pltpu.SemaphoreType.DMA,
            pltpu.SemaphoreType.DMA,
            pltpu.SemaphoreType.DMA,
            pltpu.SemaphoreType.DMA,
        ],
    )
    def k(oh_hbm, oc_hbm, row_hbm, outh_hbm, outc_hbm,
          acch, accc, idx, idxc, valh, valc,
          si0, sh0, sc0, si1, sh1, sc1):
        cid = lax.axis_index("c")
        sid = lax.axis_index("s")
        wid = sid * NC + cid
        base = wid * epw
        sems = ((si0, sh0, sc0), (si1, sh1, sc1))

        def zrow(j, c2):
            for t in range(HID // L):
                valh[0, j, pl.ds(t * L, L)] = jnp.zeros((L,), F32)
            return c2

        lax.fori_loop(0, CH, zrow, 0)
        for kk in range((rpt + CH - 1) // CH):
            sz = min(CH, rpt - kk * CH)
            pltpu.sync_copy(valh.at[0].at[pl.ds(0, sz)],
                            acch.at[pl.ds(sid * rpt + kk * CH, sz)])
        pltpu.sync_copy(valh.at[0].at[pl.ds(0, crpt)],
                        accc.at[pl.ds(sid * crpt, crpt)])
        plsc.subcore_barrier()

        def start(i, s):
            off = base + i * CH
            pltpu.async_copy(row_hbm.at[pl.ds(off, CH)], idx.at[s], sems[s][0])
            pltpu.async_copy(oh_hbm.at[pl.ds(off, CH)], valh.at[s], sems[s][1])
            pltpu.async_copy(oc_hbm.at[pl.ds(off, CH)], valc.at[s], sems[s][2])

        def finish(i, s):
            off = base + i * CH
            pltpu.make_async_copy(
                row_hbm.at[pl.ds(off, CH)], idx.at[s], sems[s][0]).wait()
            pltpu.make_async_copy(
                oh_hbm.at[pl.ds(off, CH)], valh.at[s], sems[s][1]).wait()
            pltpu.make_async_copy(
                oc_hbm.at[pl.ds(off, CH)], valc.at[s], sems[s][2]).wait()
            for g in range(CH // L):
                sl = pl.ds(g * L, L)
                idxc[s, sl] = lax.shift_right_logical(idx[s, sl], 5)
            pltpu.sync_copy(valh.at[s], acch.at[idx.at[s]], add=True)
            pltpu.sync_copy(valc.at[s], accc.at[idxc.at[s]], add=True)

        assert nchunk % 2 == 1
        start(0, 0)

        def body(g, carry):
            i0 = 2 * g
            start(i0 + 1, 1)
            finish(i0, 0)
            start(i0 + 2, 0)
            finish(i0 + 1, 1)
            return carry

        lax.fori_loop(0, (nchunk - 1) // 2, body, 0)
        finish(nchunk - 1, 0)
        plsc.subcore_barrier()
        pltpu.sync_copy(acch.at[pl.ds(sid * rpt, rpt)],
                        outh_hbm.at[cid, pl.ds(sid * rpt, rpt)])
        pltpu.sync_copy(accc.at[pl.ds(sid * crpt, crpt)],
                        outc_hbm.at[cid, pl.ds(sid * crpt, crpt)])

    return k(oh, oc, row)


def _node_mlp(h, coords, accs, deltas, wa, wb, b1, w2, b2):
    n = h.shape[0]
    na = len(accs)
    nd = len(deltas)

    def body(*refs):
        h_ref, c_ref = refs[0], refs[1]
        a_refs = refs[2:2 + na]
        d_refs = refs[2 + na:2 + na + nd]
        wa_ref, wb_ref, b1_ref, w2_ref, b2_ref = refs[2 + na + nd:-2]
        h_out, c_out = refs[-2], refs[-1]
        h_ = h_ref[:]
        agg = a_refs[0][:]
        for a in a_refs[1:]:
            agg = agg + a[:]
        z = (jnp.dot(h_, wa_ref[:], preferred_element_type=F32)
             + jnp.dot(agg, wb_ref[:], preferred_element_type=F32)
             + b1_ref[:])
        z = _silu(z)
        h_out[:] = jnp.dot(z, w2_ref[:], preferred_element_type=F32) + b2_ref[:]
        cc = c_ref[:]
        for d in d_refs:
            cc = cc + d[:]
        c_out[:] = cc

    return pl.pallas_call(
        body,
        grid=(n // NB,),
        in_specs=(
            [pl.BlockSpec((NB, HID), lambda i: (i, 0)),
             pl.BlockSpec((NB, 3), lambda i: (i, 0))]
            + [pl.BlockSpec((NB, HID), lambda i: (i, 0))] * na
            + [pl.BlockSpec((NB, 3), lambda i: (i, 0))] * nd
            + [pl.BlockSpec((HID, HID), lambda i: (0, 0)),
               pl.BlockSpec((HID, HID), lambda i: (0, 0)),
               pl.BlockSpec((1, HID), lambda i: (0, 0)),
               pl.BlockSpec((HID, HID), lambda i: (0, 0)),
               pl.BlockSpec((1, HID), lambda i: (0, 0))]
        ),
        out_specs=[
            pl.BlockSpec((NB, HID), lambda i: (i, 0)),
            pl.BlockSpec((NB, 3), lambda i: (i, 0)),
        ],
        out_shape=[
            jax.ShapeDtypeStruct((n, HID), F32),
            jax.ShapeDtypeStruct((n, 3), F32),
        ],
    )(h, coords, *accs, *deltas, wa, wb, b1, w2, b2)


def kernel(h, coords, edges, mW1, mb1, mW2, mb2, nW1, nb1, nW2, nb2,
           cW1, cb1, cW2, cb2):
    n = h.shape[0]
    e = edges.shape[1]
    row = edges[0].astype(I32)
    col = edges[1].astype(I32)

    lbn = (4.0 * (jnp.arange(n) % PACK)).astype(F32).reshape(n, 1)
    ta, tb = _build_tables(h, coords, lbn, mW1[:HID], mW1[HID:2 * HID],
                           mb1.reshape(1, HID))
    es = e // NSLAB
    accs, deltas = [], []
    for k in range(NSLAB):
        rk = lax.dynamic_slice_in_dim(row, k * es, es)
        ck = lax.dynamic_slice_in_dim(col, k * es, es)
        s = _gather_sum(ta, tb, rk, ck, es)
        oh, oc = _edge_mlp(s, mW2, mb2.reshape(1, HID), cW1,
                           cb1.reshape(1, HID // 2), cW2.reshape(1, HID // 2),
                           cb2.reshape(1, 1), mW1[2 * HID].reshape(1, HID), es)
        acch, accc = _scatter_add(oh, oc, rk, n, es)
        accs += [acch[0][:n], acch[1][:n]]
        deltas += [accc[0].reshape(-1, 4)[:n, :3],
                   accc[1].reshape(-1, 4)[:n, :3]]
    h_new, coords_new = _node_mlp(h, coords, accs, deltas,
                                  nW1[:HID], nW1[HID:], nb1.reshape(1, HID),
                                  nW2, nb2.reshape(1, HID))
    return (h_new, coords_new)
